# baseline (speedup 1.0000x reference)
import jax
import jax.numpy as jnp
from jax import lax
from jax.experimental import pallas as pl
from jax.experimental.pallas import tpu as pltpu

_TC = 16


def _gru_body(ids_ref, emb_ref, k_ref, rec_ref, b_ref, dw_ref, db_ref,
              out_ref, proj_ref, h_ref, matx_ref, seq_ref):
    i = pl.program_id(0)
    TcB = ids_ref.shape[0]
    V, D = emb_ref.shape
    H = rec_ref.shape[0]
    B = h_ref.shape[0]
    H3 = 3 * H

    @pl.when(i == 0)
    def _init():
        bias_i = b_ref[0:1, :]
        proj_ref[...] = (
            jnp.dot(emb_ref[...], k_ref[...], preferred_element_type=jnp.float32)
            + bias_i
        ).astype(jnp.bfloat16)
        h_ref[...] = jnp.zeros_like(h_ref)

    ids = ids_ref[...]
    iota = lax.broadcasted_iota(jnp.int32, (TcB, V), 1)
    onehot = (ids == iota).astype(jnp.bfloat16)
    matx_ref[...] = jnp.dot(
        onehot, proj_ref[...], preferred_element_type=jnp.float32
    ).astype(jnp.bfloat16)

    rec = rec_ref[...]
    bias_r = b_ref[1:2, :]
    Tc = TcB // B
    S = 2
    Bs = B // S

    def substep(h, mx):
        mh = jnp.dot(
            h.astype(jnp.bfloat16), rec, preferred_element_type=jnp.float32
        ) + bias_r
        z = jax.nn.sigmoid(mx[:, :H] + mh[:, :H])
        r = jax.nn.sigmoid(mx[:, H:2 * H] + mh[:, H:2 * H])
        hh = jnp.tanh(mx[:, 2 * H:] + r * mh[:, 2 * H:])
        return z * h + (1.0 - z) * hh

    hs = [h_ref[s * Bs:(s + 1) * Bs, :] for s in range(S)]
    for t in range(Tc):
        for s in range(S):
            lo = t * B + s * Bs
            mx = matx_ref[lo:lo + Bs, :].astype(jnp.float32)
            hs[s] = substep(hs[s], mx)
            seq_ref[lo:lo + Bs, :] = hs[s].astype(jnp.bfloat16)
    for s in range(S):
        h_ref[s * Bs:(s + 1) * Bs, :] = hs[s]
    out_ref[...] = (
        jnp.dot(seq_ref[...], dw_ref[...], preferred_element_type=jnp.float32)
        + db_ref[0:1, :]
    )


def kernel(inputs, emb, kernel, rec_kernel, bias, dense_w, dense_b):
    B, T = inputs.shape
    V, D = emb.shape
    H = rec_kernel.shape[0]
    H3 = 3 * H
    Tc = _TC

    ids = inputs.astype(jnp.int32).T.reshape(T * B, 1)
    b2 = jnp.zeros((8, H3), jnp.float32).at[0].set(bias[0]).at[1].set(bias[1])
    db2 = jnp.zeros((8, V), jnp.float32).at[0].set(dense_b)

    out = pl.pallas_call(
        _gru_body,
        grid=(T // Tc,),
        in_specs=[
            pl.BlockSpec((Tc * B, 1), lambda i: (i, 0)),
            pl.BlockSpec((V, D), lambda i: (0, 0)),
            pl.BlockSpec((D, H3), lambda i: (0, 0)),
            pl.BlockSpec((H, H3), lambda i: (0, 0)),
            pl.BlockSpec((8, H3), lambda i: (0, 0)),
            pl.BlockSpec((H, V), lambda i: (0, 0)),
            pl.BlockSpec((8, V), lambda i: (0, 0)),
        ],
        out_specs=pl.BlockSpec((Tc * B, V), lambda i: (i, 0)),
        out_shape=jax.ShapeDtypeStruct((T * B, V), jnp.float32),
        scratch_shapes=[
            pltpu.VMEM((V, H3), jnp.bfloat16),
            pltpu.VMEM((B, H), jnp.float32),
            pltpu.VMEM((Tc * B, H3), jnp.bfloat16),
            pltpu.VMEM((Tc * B, H), jnp.bfloat16),
        ],
        name="gru_fused",
        compiler_params=pltpu.CompilerParams(
            dimension_semantics=("arbitrary",),
        ),
    )(ids, emb, kernel, rec_kernel.astype(jnp.bfloat16), b2,
      dense_w.astype(jnp.bfloat16), db2)

    return out.reshape(T, B, V).transpose(1, 0, 2)


# speedup vs baseline: 1.4036x; 1.4036x over previous
import jax
import jax.numpy as jnp
from jax import lax
from jax.experimental import pallas as pl
from jax.experimental.pallas import tpu as pltpu

_TC = 16


def _gru_body(ids_ref, emb_ref, k_ref, rec_ref, b_ref, dw_ref, db_ref,
              out_ref, proj_ref, h_ref, matx_ref, seq_ref):
    i = pl.program_id(0)
    TcB = ids_ref.shape[0]
    V, D = emb_ref.shape
    H = rec_ref.shape[0]
    B = h_ref.shape[0]
    H3 = 3 * H

    @pl.when(i == 0)
    def _init():
        bias_i = b_ref[0:1, :]
        proj_ref[...] = (
            jnp.dot(emb_ref[...], k_ref[...], preferred_element_type=jnp.float32)
            + bias_i
        ).astype(jnp.bfloat16)
        h_ref[...] = jnp.zeros_like(h_ref)

    ids = ids_ref[...]
    iota = lax.broadcasted_iota(jnp.int32, (TcB, V), 1)
    onehot = (ids == iota).astype(jnp.bfloat16)
    matx_ref[...] = jnp.dot(
        onehot, proj_ref[...], preferred_element_type=jnp.float32
    ).astype(jnp.bfloat16)

    rec = rec_ref[...]
    bias_r = b_ref[1:2, :]
    Tc = TcB // B
    S = 1
    Bs = B // S

    def substep(h, mx):
        mh = jnp.dot(
            h.astype(jnp.bfloat16), rec, preferred_element_type=jnp.float32
        ) + bias_r
        z = jax.nn.sigmoid(mx[:, :H] + mh[:, :H])
        r = jax.nn.sigmoid(mx[:, H:2 * H] + mh[:, H:2 * H])
        hh = jnp.tanh(mx[:, 2 * H:] + r * mh[:, 2 * H:])
        return z * h + (1.0 - z) * hh

    hs = [h_ref[s * Bs:(s + 1) * Bs, :] for s in range(S)]
    for t in range(Tc):
        for s in range(S):
            lo = t * B + s * Bs
            mx = matx_ref[lo:lo + Bs, :].astype(jnp.float32)
            hs[s] = substep(hs[s], mx)
            seq_ref[lo:lo + Bs, :] = hs[s].astype(jnp.bfloat16)
    for s in range(S):
        h_ref[s * Bs:(s + 1) * Bs, :] = hs[s]
    out_ref[...] = (
        jnp.dot(seq_ref[...], dw_ref[...], preferred_element_type=jnp.float32)
        + db_ref[0:1, :]
    )


def kernel(inputs, emb, kernel, rec_kernel, bias, dense_w, dense_b):
    B, T = inputs.shape
    V, D = emb.shape
    H = rec_kernel.shape[0]
    H3 = 3 * H
    Tc = _TC

    ids = inputs.astype(jnp.int32).T.reshape(T * B, 1)
    b2 = jnp.zeros((8, H3), jnp.float32).at[0].set(bias[0]).at[1].set(bias[1])
    db2 = jnp.zeros((8, V), jnp.float32).at[0].set(dense_b)

    out = pl.pallas_call(
        _gru_body,
        grid=(T // Tc,),
        in_specs=[
            pl.BlockSpec((Tc * B, 1), lambda i: (i, 0)),
            pl.BlockSpec((V, D), lambda i: (0, 0)),
            pl.BlockSpec((D, H3), lambda i: (0, 0)),
            pl.BlockSpec((H, H3), lambda i: (0, 0)),
            pl.BlockSpec((8, H3), lambda i: (0, 0)),
            pl.BlockSpec((H, V), lambda i: (0, 0)),
            pl.BlockSpec((8, V), lambda i: (0, 0)),
        ],
        out_specs=pl.BlockSpec((Tc * B, V), lambda i: (i, 0)),
        out_shape=jax.ShapeDtypeStruct((T * B, V), jnp.float32),
        scratch_shapes=[
            pltpu.VMEM((V, H3), jnp.bfloat16),
            pltpu.VMEM((B, H), jnp.float32),
            pltpu.VMEM((Tc * B, H3), jnp.bfloat16),
            pltpu.VMEM((Tc * B, H), jnp.bfloat16),
        ],
        name="gru_fused",
        compiler_params=pltpu.CompilerParams(
            dimension_semantics=("arbitrary",),
        ),
    )(ids, emb, kernel, rec_kernel.astype(jnp.bfloat16), b2,
      dense_w.astype(jnp.bfloat16), db2)

    return out.reshape(T, B, V).transpose(1, 0, 2)
